# Initial kernel scaffold; baseline (speedup 1.0000x reference)
import jax
import jax.numpy as jnp
from jax.experimental import pallas as pl


def kernel(x, tables):
    raise NotImplementedError("write your pallas kernel here")


# speedup vs baseline: 1.0119x; 1.0119x over previous
import functools

import jax
import jax.numpy as jnp
from jax import lax
from jax.experimental import pallas as pl
from jax.experimental.pallas import tpu as pltpu
from jax.experimental.pallas import tpu_sc as plsc

F = 26
V = 100000
D = 32
B = 4096

NC = 2
NS = 16
NW = NC * NS
ROWS = B * F
RPW = ROWS // NW
CHUNK = 128
NCH = RPW // CHUNK
VPC = CHUNK // 16


@functools.partial(
    pl.kernel,
    mesh=plsc.VectorSubcoreMesh(core_axis_name="c", subcore_axis_name="s"),
    out_type=jax.ShapeDtypeStruct((NW, RPW, D), jnp.float32),
    scratch_types=[
        pltpu.VMEM((NCH, CHUNK), jnp.int32),
        pltpu.VMEM((RPW, D), jnp.float32),
        pltpu.SemaphoreType.DMA,
    ],
    compiler_params=pltpu.CompilerParams(use_tc_tiling_on_sc=False),
)
def _embed_gather(x_hbm, tab_hbm, out_hbm, idx_v, rows_v, sem):
    wid = lax.axis_index("s") * NC + lax.axis_index("c")

    pltpu.sync_copy(x_hbm.at[wid], idx_v)

    lane = lax.broadcasted_iota(jnp.int32, (16,), 0)

    def fire(j, carry):
        for v in range(VPC):
            pos = j * CHUNK + v * 16 + lane
            fld = lax.rem(pos, F)
            idx_v[j, pl.ds(v * 16, 16)] = (
                idx_v[j, pl.ds(v * 16, 16)] + fld * V
            )
        pltpu.make_async_copy(
            tab_hbm.at[idx_v.at[j]],
            rows_v.at[pl.ds(j * CHUNK, CHUNK)],
            sem,
        ).start()
        return carry

    lax.fori_loop(0, NCH, fire, 0)

    def drain(j, carry):
        pltpu.make_async_copy(
            tab_hbm.at[idx_v.at[j]],
            rows_v.at[pl.ds(j * CHUNK, CHUNK)],
            sem,
        ).wait()
        return carry

    lax.fori_loop(0, NCH, drain, 0)

    pltpu.sync_copy(rows_v, out_hbm.at[wid])


def kernel(x, tables):
    x3 = x.astype(jnp.int32).reshape(NW, NCH, CHUNK)
    tab2 = tables.reshape(F * V, D)
    out = _embed_gather(x3, tab2)
    return out.reshape(B, F, D)


# speedup vs baseline: 6.0074x; 5.9366x over previous
import functools

import jax
import jax.numpy as jnp
from jax import lax
from jax.experimental import pallas as pl
from jax.experimental.pallas import tpu as pltpu
from jax.experimental.pallas import tpu_sc as plsc

F = 26
V = 100000
D = 32
B = 4096

NC = 2
NS = 16
NW = NC * NS
UNITS = F * D
UPW = UNITS // NW
NG = B // 16


@functools.partial(
    pl.kernel,
    mesh=plsc.VectorSubcoreMesh(core_axis_name="c", subcore_axis_name="s"),
    out_type=jax.ShapeDtypeStruct((F, D, B), jnp.float32),
    scratch_types=[
        pltpu.VMEM((B,), jnp.int32),
        pltpu.VMEM((V,), jnp.float32),
        pltpu.VMEM((B,), jnp.float32),
        pltpu.SemaphoreType.DMA,
    ],
    compiler_params=pltpu.CompilerParams(
        use_tc_tiling_on_sc=True, needs_layout_passes=False
    ),
)
def _embed_gather(x_hbm, tab_hbm, out_hbm, x_v, tab_v, out_v, sem):
    wid = lax.axis_index("s") * NC + lax.axis_index("c")
    u0 = wid * UPW

    def unit(k, carry):
        u = u0 + k
        f = u // D
        d = u - f * D
        pltpu.sync_copy(x_hbm.at[f], x_v)
        pltpu.sync_copy(tab_hbm.at[f, d], tab_v)

        def gather(i, c2):
            idx16 = x_v[pl.ds(i * 16, 16)]
            out_v[pl.ds(i * 16, 16)] = plsc.load_gather(tab_v, [idx16])
            return c2

        lax.fori_loop(0, NG, gather, 0)
        pltpu.sync_copy(out_v, out_hbm.at[f, d])
        return carry

    lax.fori_loop(0, UPW, unit, 0)


def kernel(x, tables):
    x_t = x.astype(jnp.int32).T
    tab_t = tables.transpose(0, 2, 1)
    out_t = _embed_gather(x_t, tab_t)
    return out_t.transpose(2, 0, 1)


# speedup vs baseline: 6.4446x; 1.0728x over previous
import functools

import jax
import jax.numpy as jnp
from jax import lax
from jax.experimental import pallas as pl
from jax.experimental.pallas import tpu as pltpu
from jax.experimental.pallas import tpu_sc as plsc

F = 26
V = 100000
D = 32
B = 4096

NC = 2
NS = 16
NW = NC * NS
UNITS = F * D
UPW = UNITS // NW
NG = B // 16
H0 = 49920
H1 = V - H0


@functools.partial(
    pl.kernel,
    mesh=plsc.VectorSubcoreMesh(core_axis_name="c", subcore_axis_name="s"),
    out_type=jax.ShapeDtypeStruct((F, D, B), jnp.float32),
    scratch_types=[
        pltpu.VMEM((B,), jnp.int32),
        pltpu.VMEM((H0,), jnp.float32),
        pltpu.VMEM((H1,), jnp.float32),
        pltpu.VMEM((2, B), jnp.float32),
        pltpu.SemaphoreType.DMA,
        pltpu.SemaphoreType.DMA,
    ],
    compiler_params=pltpu.CompilerParams(
        use_tc_tiling_on_sc=True, needs_layout_passes=False
    ),
)
def _embed_gather(
    x_hbm, tab_hbm, out_hbm, x_v, rowa_v, rowb_v, out_v, sem_r, sem_o
):
    wid = lax.axis_index("s") * NC + lax.axis_index("c")
    u0 = wid * UPW
    lane = lax.broadcasted_iota(jnp.int32, (16,), 0)

    def fd(u):
        f = u // D
        return f, u - f * D

    def fire_piece(k, h):
        f, d = fd(u0 + k)
        src = tab_hbm.at[f, d]
        if h == 0:
            pltpu.make_async_copy(src.at[pl.ds(0, H0)], rowa_v, sem_r).start()
        else:
            pltpu.make_async_copy(src.at[pl.ds(H0, H1)], rowb_v, sem_r).start()

    def wait_piece(h):
        src = tab_hbm.at[0, 0]
        if h == 0:
            pltpu.make_async_copy(src.at[pl.ds(0, H0)], rowa_v, sem_r).wait()
        else:
            pltpu.make_async_copy(src.at[pl.ds(H0, H1)], rowb_v, sem_r).wait()

    def gather_pass(row_ref, base, size, oslot16):
        def gather(i, c2):
            pos = i * 16 + lane
            idx16 = x_v[pl.ds(i * 16, 16)]
            rel = idx16 - base
            inb = (rel >= 0) & (rel < size)
            relc = lax.max(0, lax.min(rel, size - 1))
            val = plsc.load_gather(row_ref, [relc])
            plsc.store_scatter(out_v, [oslot16, pos], val, mask=inb)
            return c2

        lax.fori_loop(0, NG, gather, 0)

    fire_piece(0, 0)
    fire_piece(0, 1)

    def step(k, prev_f):
        oslot = k % 2
        f, d = fd(u0 + k)
        oslot16 = jnp.full((16,), oslot, dtype=jnp.int32)

        @pl.when(f != prev_f)
        def _():
            pltpu.sync_copy(x_hbm.at[f], x_v)

        @pl.when(k >= 2)
        def _():
            pltpu.make_async_copy(out_v.at[0], out_hbm.at[0, 0], sem_o).wait()

        wait_piece(0)
        gather_pass(rowa_v, 0, H0, oslot16)

        @pl.when(k + 1 < UPW)
        def _():
            fire_piece(k + 1, 0)

        wait_piece(1)
        gather_pass(rowb_v, H0, H1, oslot16)

        @pl.when(k + 1 < UPW)
        def _():
            fire_piece(k + 1, 1)

        pltpu.make_async_copy(out_v.at[oslot], out_hbm.at[f, d], sem_o).start()
        return f

    lax.fori_loop(0, UPW, step, jnp.int32(-1))

    pltpu.make_async_copy(out_v.at[0], out_hbm.at[0, 0], sem_o).wait()
    pltpu.make_async_copy(out_v.at[0], out_hbm.at[0, 0], sem_o).wait()


def kernel(x, tables):
    x_t = x.astype(jnp.int32).T
    tab_t = tables.transpose(0, 2, 1)
    out_t = _embed_gather(x_t, tab_t)
    return out_t.transpose(2, 0, 1)


# speedup vs baseline: 6.5021x; 1.0089x over previous
import functools

import jax
import jax.numpy as jnp
from jax import lax
from jax.experimental import pallas as pl
from jax.experimental.pallas import tpu as pltpu
from jax.experimental.pallas import tpu_sc as plsc

F = 26
V = 100000
D = 32
B = 4096

NC = 2
NS = 16
NW = NC * NS
UNITS = F * D
UPW = UNITS // NW
NG = B // 16
H0 = 49920
H1 = V - H0


@functools.partial(
    pl.kernel,
    mesh=plsc.VectorSubcoreMesh(core_axis_name="c", subcore_axis_name="s"),
    out_type=jax.ShapeDtypeStruct((F, D, B), jnp.float32),
    scratch_types=[
        pltpu.VMEM((B,), jnp.int32),
        pltpu.VMEM((H0,), jnp.float32),
        pltpu.VMEM((H1,), jnp.float32),
        pltpu.VMEM((2, B), jnp.float32),
        pltpu.SemaphoreType.DMA,
        pltpu.SemaphoreType.DMA,
    ],
    compiler_params=pltpu.CompilerParams(
        use_tc_tiling_on_sc=True, needs_layout_passes=False
    ),
)
def _embed_gather(
    x_hbm, tab_hbm, out_hbm, x_v, rowa_v, rowb_v, out_v, sem_r, sem_o
):
    wid = lax.axis_index("s") * NC + lax.axis_index("c")
    u0 = wid * UPW
    lane = lax.broadcasted_iota(jnp.int32, (16,), 0)

    def fd(u):
        f = u // D
        return f, u - f * D

    def fire_piece(k, h):
        f, d = fd(u0 + k)
        src = tab_hbm.at[f, d]
        if h == 0:
            pltpu.make_async_copy(src.at[pl.ds(0, H0)], rowa_v, sem_r).start()
        else:
            pltpu.make_async_copy(src.at[pl.ds(H0, H1)], rowb_v, sem_r).start()

    def wait_piece(h):
        src = tab_hbm.at[0, 0]
        if h == 0:
            pltpu.make_async_copy(src.at[pl.ds(0, H0)], rowa_v, sem_r).wait()
        else:
            pltpu.make_async_copy(src.at[pl.ds(H0, H1)], rowb_v, sem_r).wait()

    def gather_pass(row_ref, base, size, oslot16):
        def gather(i, c2):
            pos = i * 16 + lane
            idx16 = x_v[pl.ds(i * 16, 16)]
            rel = idx16 - base
            inb = (rel >= 0) & (rel < size)
            relc = lax.max(0, lax.min(rel, size - 1))
            val = plsc.load_gather(row_ref, [relc])
            plsc.store_scatter(out_v, [oslot16, pos], val, mask=inb)
            return c2

        lax.fori_loop(0, NG, gather, 0, unroll=8)

    fire_piece(0, 0)
    fire_piece(0, 1)

    def step(k, prev_f):
        oslot = k % 2
        f, d = fd(u0 + k)
        oslot16 = jnp.full((16,), oslot, dtype=jnp.int32)

        @pl.when(f != prev_f)
        def _():
            pltpu.sync_copy(x_hbm.at[f], x_v)

        @pl.when(k >= 2)
        def _():
            pltpu.make_async_copy(out_v.at[0], out_hbm.at[0, 0], sem_o).wait()

        wait_piece(0)
        gather_pass(rowa_v, 0, H0, oslot16)

        @pl.when(k + 1 < UPW)
        def _():
            fire_piece(k + 1, 0)

        wait_piece(1)
        gather_pass(rowb_v, H0, H1, oslot16)

        @pl.when(k + 1 < UPW)
        def _():
            fire_piece(k + 1, 1)

        pltpu.make_async_copy(out_v.at[oslot], out_hbm.at[f, d], sem_o).start()
        return f

    lax.fori_loop(0, UPW, step, jnp.int32(-1))

    pltpu.make_async_copy(out_v.at[0], out_hbm.at[0, 0], sem_o).wait()
    pltpu.make_async_copy(out_v.at[0], out_hbm.at[0, 0], sem_o).wait()


def kernel(x, tables):
    x_t = x.astype(jnp.int32).T
    tab_t = tables.transpose(0, 2, 1)
    out_t = _embed_gather(x_t, tab_t)
    return out_t.transpose(2, 0, 1)


# speedup vs baseline: 6.5047x; 1.0004x over previous
import functools

import jax
import jax.numpy as jnp
from jax import lax
from jax.experimental import pallas as pl
from jax.experimental.pallas import tpu as pltpu
from jax.experimental.pallas import tpu_sc as plsc

F = 26
V = 100000
D = 32
B = 4096

NC = 2
NS = 16
NW = NC * NS
UNITS = F * D
UPW = UNITS // NW
NG = B // 16
H0 = 49920
H1 = V - H0


@functools.partial(
    pl.kernel,
    mesh=plsc.VectorSubcoreMesh(core_axis_name="c", subcore_axis_name="s"),
    out_type=jax.ShapeDtypeStruct((F, D, B), jnp.float32),
    scratch_types=[
        pltpu.VMEM((B,), jnp.int32),
        pltpu.VMEM((H0,), jnp.float32),
        pltpu.VMEM((H1,), jnp.float32),
        pltpu.VMEM((2, B), jnp.float32),
        pltpu.SemaphoreType.DMA,
        pltpu.SemaphoreType.DMA,
    ],
    compiler_params=pltpu.CompilerParams(
        use_tc_tiling_on_sc=True, needs_layout_passes=False,
        disable_bounds_checks=True, disable_semaphore_checks=True
    ),
)
def _embed_gather(
    x_hbm, tab_hbm, out_hbm, x_v, rowa_v, rowb_v, out_v, sem_r, sem_o
):
    wid = lax.axis_index("s") * NC + lax.axis_index("c")
    u0 = wid * UPW
    lane = lax.broadcasted_iota(jnp.int32, (16,), 0)

    def fd(u):
        f = u // D
        return f, u - f * D

    def fire_piece(k, h):
        f, d = fd(u0 + k)
        src = tab_hbm.at[f, d]
        if h == 0:
            pltpu.make_async_copy(src.at[pl.ds(0, H0)], rowa_v, sem_r).start()
        else:
            pltpu.make_async_copy(src.at[pl.ds(H0, H1)], rowb_v, sem_r).start()

    def wait_piece(h):
        src = tab_hbm.at[0, 0]
        if h == 0:
            pltpu.make_async_copy(src.at[pl.ds(0, H0)], rowa_v, sem_r).wait()
        else:
            pltpu.make_async_copy(src.at[pl.ds(H0, H1)], rowb_v, sem_r).wait()

    def gather_pass(row_ref, base, size, oslot16):
        def gather(i, c2):
            pos = i * 16 + lane
            idx16 = x_v[pl.ds(i * 16, 16)]
            rel = idx16 - base
            inb = (rel >= 0) & (rel < size)
            relc = lax.max(0, lax.min(rel, size - 1))
            val = plsc.load_gather(row_ref, [relc])
            plsc.store_scatter(out_v, [oslot16, pos], val, mask=inb)
            return c2

        lax.fori_loop(0, NG, gather, 0, unroll=8)

    fire_piece(0, 0)
    fire_piece(0, 1)

    def step(k, prev_f):
        oslot = k % 2
        f, d = fd(u0 + k)
        oslot16 = jnp.full((16,), oslot, dtype=jnp.int32)

        @pl.when(f != prev_f)
        def _():
            pltpu.sync_copy(x_hbm.at[f], x_v)

        @pl.when(k >= 2)
        def _():
            pltpu.make_async_copy(out_v.at[0], out_hbm.at[0, 0], sem_o).wait()

        wait_piece(0)
        gather_pass(rowa_v, 0, H0, oslot16)

        @pl.when(k + 1 < UPW)
        def _():
            fire_piece(k + 1, 0)

        wait_piece(1)
        gather_pass(rowb_v, H0, H1, oslot16)

        @pl.when(k + 1 < UPW)
        def _():
            fire_piece(k + 1, 1)

        pltpu.make_async_copy(out_v.at[oslot], out_hbm.at[f, d], sem_o).start()
        return f

    lax.fori_loop(0, UPW, step, jnp.int32(-1))

    pltpu.make_async_copy(out_v.at[0], out_hbm.at[0, 0], sem_o).wait()
    pltpu.make_async_copy(out_v.at[0], out_hbm.at[0, 0], sem_o).wait()


def kernel(x, tables):
    x_t = x.astype(jnp.int32).T
    tab_t = tables.transpose(0, 2, 1)
    out_t = _embed_gather(x_t, tab_t)
    return out_t.transpose(2, 0, 1)
